# baseline (speedup 1.0000x reference)
import functools

import jax
import jax.numpy as jnp
from jax import lax
from jax.experimental import pallas as pl
from jax.experimental.pallas import tpu as pltpu
from jax.experimental.pallas import tpu_sc as plsc

NUM_L0, NUM_L1, NUM_L2 = 4, 16, 16
P = NUM_L0 + NUM_L1 + NUM_L2 + 1
B, T, D = 64, 256, 128
S = T * P
N = B * S

T_L0_BASE = 0
T_L1_BASE = 16
T_L2_BASE = 80
T_ACT_BASE = 144
T_ROWS = 153
T_PAD = 256
BIAS_PAD = 40

NC, NS = 2, 16
NW = NC * NS
SW = S // NW
NB = 2
NG = B // NB
H1, H2 = 152, 144


def _build_t_body(l0_ref, l1_ref, l2_ref, act_ref, lvl_ref, patch_ref,
                  t_ref, bias_ref):
    t_ref[T_L0_BASE:T_L1_BASE, :] = l0_ref[...]
    t_ref[T_L1_BASE:T_L2_BASE, :] = l1_ref[...]
    t_ref[T_L2_BASE:T_ACT_BASE, :] = l2_ref[...]
    t_ref[T_ACT_BASE:T_ROWS, :] = act_ref[...]
    t_ref[T_ROWS:T_PAD, :] = jnp.zeros((T_PAD - T_ROWS, D), jnp.float32)
    for p in range(NUM_L0):
        bias_ref[p : p + 1, :] = lvl_ref[0:1, :] + patch_ref[p : p + 1, :]
    for j in range(NUM_L1):
        bias_ref[NUM_L0 + j : NUM_L0 + j + 1, :] = (
            lvl_ref[1:2, :] + patch_ref[j : j + 1, :]
        )
    for j in range(NUM_L2):
        bias_ref[20 + j : 21 + j, :] = lvl_ref[2:3, :] + patch_ref[j : j + 1, :]
    bias_ref[36:37, :] = lvl_ref[3:4, :]
    bias_ref[37:BIAS_PAD, :] = jnp.zeros((BIAS_PAD - 37, D), jnp.float32)


def _build_idx_body(t0_ref, t1_ref, t2_ref, act_ref, out_ref):
    out_ref[:, :, 0:NUM_L0] = t0_ref[...] + T_L0_BASE
    out_ref[:, :, NUM_L0 : NUM_L0 + NUM_L1] = t1_ref[...] + T_L1_BASE
    out_ref[:, :, NUM_L0 + NUM_L1 : P - 1] = t2_ref[...] + T_L2_BASE
    out_ref[:, :, P - 1 : P] = act_ref[...] + T_ACT_BASE


def _sc_body(t_hbm, idx_hbm, pos_hbm, bias_hbm, out_hbm, t_sh,
             bias_v, pos_v, idx0, idx1, idx2, idx3, wk0, wk1,
             is0, is1, is2, is3, gsa0, gsb0, gsa1, gsb1,
             ssa0, ssb0, ssa1, ssb1):
    cid = lax.axis_index("c")
    sid = lax.axis_index("s")
    wid = sid * NC + cid
    s0 = wid * SW

    idxs = (idx0, idx1, idx2, idx3)
    works = (wk0, wk1)
    isems = (is0, is1, is2, is3)
    gsems = ((gsa0, gsb0), (gsa1, gsb1))
    ssems = ((ssa0, ssb0), (ssa1, ssb1))
    HALVES = (((0, 128), (128, 24)), ((152, 104), (256, 40)))
    HOFF = (0, H1)
    HLEN = (H1, H2)

    tr = T_PAD // NS
    st0 = pltpu.async_copy(
        t_hbm.at[pl.ds(sid * tr, tr)], t_sh.at[pl.ds(sid * tr, tr)], is0)
    st1 = pltpu.async_copy(pos_hbm.at[pl.ds(s0, SW)], pos_v, is1)
    st2 = pltpu.async_copy(bias_hbm, bias_v, gsa0)
    st0.wait()
    st1.wait()
    st2.wait()
    plsc.subcore_barrier()

    @plsc.parallel_loop(0, SW, 1, unroll=4)
    def _(i):
        m = lax.rem(i, P)
        for v in range(D // 16):
            sl = pl.ds(v * 16, 16)
            pos_v[i, sl] = pos_v[i, sl] + bias_v[m, sl]

    def issue_idx(c, j):
        pltpu.async_copy(idx_hbm.at[pl.ds(c * S + s0, SW)], idxs[j], isems[j])

    def wait_idx(j):
        pltpu.make_async_copy(
            idx_hbm.at[pl.ds(0, SW)], idxs[j], isems[j]).wait()

    def issue_gather_half(b, h, j):
        for (o, n) in HALVES[h]:
            pltpu.async_copy(
                t_sh.at[idxs[j].at[pl.ds(o, n)]],
                works[b].at[pl.ds(o, n)], gsems[b][h])

    def wait_gather_half(b, h, j):
        for (o, n) in HALVES[h]:
            pltpu.make_async_copy(
                t_sh.at[idxs[j].at[pl.ds(o, n)]],
                works[b].at[pl.ds(o, n)], gsems[b][h]).wait()

    def issue_store_half(b, h, c):
        off = c * S + s0 + HOFF[h]
        pltpu.async_copy(
            works[b].at[pl.ds(HOFF[h], HLEN[h])],
            out_hbm.at[pl.ds(off, HLEN[h])], ssems[b][h])

    def wait_store_half(b, h):
        pltpu.make_async_copy(
            works[b].at[pl.ds(HOFF[h], HLEN[h])],
            out_hbm.at[pl.ds(0, HLEN[h])], ssems[b][h]).wait()

    def add_pos(b, lo, hi):
        @plsc.parallel_loop(lo, hi, 1, unroll=16)
        def _(i):
            for v in range(D // 16):
                sl = pl.ds(v * 16, 16)
                works[b][i, sl] = works[b][i, sl] + pos_v[i, sl]

    for c in range(3):
        issue_idx(c, c)
    for b in range(2):
        wait_idx(b)
        issue_gather_half(b, 0, b)
        issue_gather_half(b, 1, b)

    def outer(gg, _):
        for par in range(2):
            g = gg * 2 + par
            c0 = g * NB
            j0 = (2 * par) % 4
            j1 = (2 * par + 1) % 4
            j2 = (2 * par + 2) % 4
            j3 = (2 * par + 3) % 4

            wait_gather_half(0, 0, j0)

            @pl.when(g > 0)
            def _():
                wait_store_half(1, 0)
                wait_idx(j1)
                issue_gather_half(1, 0, j1)

            @pl.when(c0 + 3 < B)
            def _():
                issue_idx(c0 + 3, j3)

            add_pos(0, 0, H1)
            issue_store_half(0, 0, c0)

            wait_gather_half(0, 1, j0)

            @pl.when(g > 0)
            def _():
                wait_store_half(1, 1)
                issue_gather_half(1, 1, j1)

            add_pos(0, H1, SW)
            issue_store_half(0, 1, c0)

            wait_gather_half(1, 0, j1)

            @pl.when(c0 + 2 < B)
            def _():
                wait_store_half(0, 0)
                wait_idx(j2)
                issue_gather_half(0, 0, j2)

            @pl.when(c0 + 4 < B)
            def _():
                issue_idx(c0 + 4, j0)

            add_pos(1, 0, H1)
            issue_store_half(1, 0, c0 + 1)

            wait_gather_half(1, 1, j1)

            @pl.when(c0 + 2 < B)
            def _():
                wait_store_half(0, 1)
                issue_gather_half(0, 1, j2)

            add_pos(1, H1, SW)
            issue_store_half(1, 1, c0 + 1)

        return 0

    lax.fori_loop(0, NG // 2, outer, 0)

    wait_store_half(0, 0)
    wait_store_half(0, 1)
    wait_store_half(1, 0)
    wait_store_half(1, 1)


def _make_sc_gather():
    return pl.kernel(
        _sc_body,
        out_type=jax.ShapeDtypeStruct((N, D), jnp.float32),
        mesh=plsc.VectorSubcoreMesh(
            core_axis_name="c", subcore_axis_name="s",
            num_cores=NC, num_subcores=NS,
        ),
        scratch_types=[
            pltpu.VMEM_SHARED((T_PAD, D), jnp.float32),
            pltpu.VMEM((BIAS_PAD, D), jnp.float32),
            pltpu.VMEM((SW, D), jnp.float32),
        ]
        + [pltpu.VMEM((SW,), jnp.int32) for _ in range(4)]
        + [pltpu.VMEM((SW, D), jnp.float32) for _ in range(NB)]
        + [pltpu.SemaphoreType.DMA for _ in range(12)],
    )


def kernel(tokens_l0, tokens_l1, tokens_l2, actions, l0_embed, l1_embed,
           l2_embed, act_embed, level_embed, patch_embed, pos_embed):
    t_table, bias = pl.pallas_call(
        _build_t_body,
        out_shape=[
            jax.ShapeDtypeStruct((T_PAD, D), jnp.float32),
            jax.ShapeDtypeStruct((BIAS_PAD, D), jnp.float32),
        ],
    )(l0_embed, l1_embed, l2_embed, act_embed, level_embed, patch_embed)

    bb = 8
    idx = pl.pallas_call(
        _build_idx_body,
        grid=(B // bb,),
        in_specs=[
            pl.BlockSpec((bb, T, NUM_L0), lambda i: (i, 0, 0)),
            pl.BlockSpec((bb, T, NUM_L1), lambda i: (i, 0, 0)),
            pl.BlockSpec((bb, T, NUM_L2), lambda i: (i, 0, 0)),
            pl.BlockSpec((bb, T, 1), lambda i: (i, 0, 0)),
        ],
        out_specs=pl.BlockSpec((bb, T, P), lambda i: (i, 0, 0)),
        out_shape=jax.ShapeDtypeStruct((B, T, P), jnp.int32),
    )(
        tokens_l0.astype(jnp.int32),
        tokens_l1.astype(jnp.int32),
        tokens_l2.astype(jnp.int32),
        actions.astype(jnp.int32).reshape(B, T, 1),
    )

    out = _make_sc_gather()(t_table, idx.reshape(N), pos_embed[:S], bias)
    return out.reshape(B, S, D)


# speedup vs baseline: 1.2577x; 1.2577x over previous
import functools

import jax
import jax.numpy as jnp
from jax import lax
from jax.experimental import pallas as pl
from jax.experimental.pallas import tpu as pltpu
from jax.experimental.pallas import tpu_sc as plsc

NUM_L0, NUM_L1, NUM_L2 = 4, 16, 16
P = NUM_L0 + NUM_L1 + NUM_L2 + 1
B, T, D = 64, 256, 128
S = T * P
N = B * S

T_L0_BASE = 0
T_L1_BASE = 16
T_L2_BASE = 80
T_ACT_BASE = 144
T_ROWS = 153
T_PAD = 256
BIAS_PAD = 40

NC, NS = 2, 16
NW = NC * NS
SW = S // NW
NB = 2
NG = B // NB
H1, H2 = 152, 144


def _build_t_body(l0_ref, l1_ref, l2_ref, act_ref, lvl_ref, patch_ref,
                  t_ref, bias_ref):
    t_ref[T_L0_BASE:T_L1_BASE, :] = l0_ref[...]
    t_ref[T_L1_BASE:T_L2_BASE, :] = l1_ref[...]
    t_ref[T_L2_BASE:T_ACT_BASE, :] = l2_ref[...]
    t_ref[T_ACT_BASE:T_ROWS, :] = act_ref[...]
    t_ref[T_ROWS:T_PAD, :] = jnp.zeros((T_PAD - T_ROWS, D), jnp.float32)
    for p in range(NUM_L0):
        bias_ref[p : p + 1, :] = lvl_ref[0:1, :] + patch_ref[p : p + 1, :]
    for j in range(NUM_L1):
        bias_ref[NUM_L0 + j : NUM_L0 + j + 1, :] = (
            lvl_ref[1:2, :] + patch_ref[j : j + 1, :]
        )
    for j in range(NUM_L2):
        bias_ref[20 + j : 21 + j, :] = lvl_ref[2:3, :] + patch_ref[j : j + 1, :]
    bias_ref[36:37, :] = lvl_ref[3:4, :]
    bias_ref[37:BIAS_PAD, :] = jnp.zeros((BIAS_PAD - 37, D), jnp.float32)


def _build_idx_body(t0_ref, t1_ref, t2_ref, act_ref, out_ref):
    out_ref[:, :, 0:NUM_L0] = t0_ref[...] + T_L0_BASE
    out_ref[:, :, NUM_L0 : NUM_L0 + NUM_L1] = t1_ref[...] + T_L1_BASE
    out_ref[:, :, NUM_L0 + NUM_L1 : P - 1] = t2_ref[...] + T_L2_BASE
    out_ref[:, :, P - 1 : P] = act_ref[...] + T_ACT_BASE


def _sc_body(t_hbm, idx_hbm, pos_hbm, bias_hbm, out_hbm, t_sh,
             bias_v, pos_v, idx0, idx1, idx2, idx3, wk0, wk1,
             is0, is1, is2, is3, gsa0, gsb0, gsa1, gsb1,
             ssa0, ssb0, ssa1, ssb1):
    cid = lax.axis_index("c")
    sid = lax.axis_index("s")
    wid = sid * NC + cid
    s0 = wid * SW

    idxs = (idx0, idx1, idx2, idx3)
    works = (wk0, wk1)
    isems = (is0, is1, is2, is3)
    gsems = ((gsa0, gsb0), (gsa1, gsb1))
    ssems = ((ssa0, ssb0), (ssa1, ssb1))
    HALVES = (((0, 128), (128, 24)), ((152, 104), (256, 40)))
    HOFF = (0, H1)
    HLEN = (H1, H2)

    tr = T_PAD // NS
    st0 = pltpu.async_copy(
        t_hbm.at[pl.ds(sid * tr, tr)], t_sh.at[pl.ds(sid * tr, tr)], is0)
    st1 = pltpu.async_copy(pos_hbm.at[pl.ds(s0, SW)], pos_v, is1)
    st2 = pltpu.async_copy(bias_hbm, bias_v, gsa0)
    st0.wait()
    st1.wait()
    st2.wait()
    plsc.subcore_barrier()

    @plsc.parallel_loop(0, SW, 1, unroll=4)
    def _(i):
        m = lax.rem(i, P)
        for v in range(D // 16):
            sl = pl.ds(v * 16, 16)
            pos_v[i, sl] = pos_v[i, sl] + bias_v[m, sl]

    def issue_idx(c, j):
        pltpu.async_copy(idx_hbm.at[pl.ds(c * S + s0, SW)], idxs[j], isems[j])

    def wait_idx(j):
        pltpu.make_async_copy(
            idx_hbm.at[pl.ds(0, SW)], idxs[j], isems[j]).wait()

    def issue_gather_half(b, h, j):
        for (o, n) in HALVES[h]:
            pltpu.async_copy(
                t_sh.at[idxs[j].at[pl.ds(o, n)]],
                works[b].at[pl.ds(o, n)], gsems[b][h])

    def wait_gather_half(b, h, j):
        for (o, n) in HALVES[h]:
            pltpu.make_async_copy(
                t_sh.at[idxs[j].at[pl.ds(o, n)]],
                works[b].at[pl.ds(o, n)], gsems[b][h]).wait()

    def issue_store_half(b, h, c):
        off = c * S + s0 + HOFF[h]
        pltpu.async_copy(
            works[b].at[pl.ds(HOFF[h], HLEN[h])],
            out_hbm.at[pl.ds(off, HLEN[h])], ssems[b][h])

    def wait_store_half(b, h):
        pltpu.make_async_copy(
            works[b].at[pl.ds(HOFF[h], HLEN[h])],
            out_hbm.at[pl.ds(0, HLEN[h])], ssems[b][h]).wait()

    def add_pos(b, lo, hi):
        @plsc.parallel_loop(lo, hi, 1, unroll=8)
        def _(i):
            for v in range(D // 16):
                sl = pl.ds(v * 16, 16)
                works[b][i, sl] = works[b][i, sl] + pos_v[i, sl]

    for c in range(3):
        issue_idx(c, c)
    for b in range(2):
        wait_idx(b)
        issue_gather_half(b, 0, b)
        issue_gather_half(b, 1, b)

    def outer(gg, _):
        for par in range(2):
            g = gg * 2 + par
            c0 = g * NB
            j0 = (2 * par) % 4
            j1 = (2 * par + 1) % 4
            j2 = (2 * par + 2) % 4
            j3 = (2 * par + 3) % 4

            wait_gather_half(0, 0, j0)

            @pl.when(g > 0)
            def _():
                wait_store_half(1, 0)
                wait_idx(j1)
                issue_gather_half(1, 0, j1)

            @pl.when(c0 + 3 < B)
            def _():
                issue_idx(c0 + 3, j3)

            add_pos(0, 0, H1)
            issue_store_half(0, 0, c0)

            wait_gather_half(0, 1, j0)

            @pl.when(g > 0)
            def _():
                wait_store_half(1, 1)
                issue_gather_half(1, 1, j1)

            add_pos(0, H1, SW)
            issue_store_half(0, 1, c0)

            wait_gather_half(1, 0, j1)

            @pl.when(c0 + 2 < B)
            def _():
                wait_store_half(0, 0)
                wait_idx(j2)
                issue_gather_half(0, 0, j2)

            @pl.when(c0 + 4 < B)
            def _():
                issue_idx(c0 + 4, j0)

            add_pos(1, 0, H1)
            issue_store_half(1, 0, c0 + 1)

            wait_gather_half(1, 1, j1)

            @pl.when(c0 + 2 < B)
            def _():
                wait_store_half(0, 1)
                issue_gather_half(0, 1, j2)

            add_pos(1, H1, SW)
            issue_store_half(1, 1, c0 + 1)

        return 0

    lax.fori_loop(0, NG // 2, outer, 0)

    wait_store_half(0, 0)
    wait_store_half(0, 1)
    wait_store_half(1, 0)
    wait_store_half(1, 1)


def _make_sc_gather():
    return pl.kernel(
        _sc_body,
        out_type=jax.ShapeDtypeStruct((N, D), jnp.float32),
        mesh=plsc.VectorSubcoreMesh(
            core_axis_name="c", subcore_axis_name="s",
            num_cores=NC, num_subcores=NS,
        ),
        scratch_types=[
            pltpu.VMEM_SHARED((T_PAD, D), jnp.float32),
            pltpu.VMEM((BIAS_PAD, D), jnp.float32),
            pltpu.VMEM((SW, D), jnp.float32),
        ]
        + [pltpu.VMEM((SW,), jnp.int32) for _ in range(4)]
        + [pltpu.VMEM((SW, D), jnp.float32) for _ in range(NB)]
        + [pltpu.SemaphoreType.DMA for _ in range(12)],
    )


def kernel(tokens_l0, tokens_l1, tokens_l2, actions, l0_embed, l1_embed,
           l2_embed, act_embed, level_embed, patch_embed, pos_embed):
    t_table, bias = pl.pallas_call(
        _build_t_body,
        out_shape=[
            jax.ShapeDtypeStruct((T_PAD, D), jnp.float32),
            jax.ShapeDtypeStruct((BIAS_PAD, D), jnp.float32),
        ],
    )(l0_embed, l1_embed, l2_embed, act_embed, level_embed, patch_embed)

    bb = 8
    idx = pl.pallas_call(
        _build_idx_body,
        grid=(B // bb,),
        in_specs=[
            pl.BlockSpec((bb, T, NUM_L0), lambda i: (i, 0, 0)),
            pl.BlockSpec((bb, T, NUM_L1), lambda i: (i, 0, 0)),
            pl.BlockSpec((bb, T, NUM_L2), lambda i: (i, 0, 0)),
            pl.BlockSpec((bb, T, 1), lambda i: (i, 0, 0)),
        ],
        out_specs=pl.BlockSpec((bb, T, P), lambda i: (i, 0, 0)),
        out_shape=jax.ShapeDtypeStruct((B, T, P), jnp.int32),
    )(
        tokens_l0.astype(jnp.int32),
        tokens_l1.astype(jnp.int32),
        tokens_l2.astype(jnp.int32),
        actions.astype(jnp.int32).reshape(B, T, 1),
    )

    out = _make_sc_gather()(t_table, idx.reshape(N), pos_embed[:S], bias)
    return out.reshape(B, S, D)


# speedup vs baseline: 1.2689x; 1.0089x over previous
import functools

import jax
import jax.numpy as jnp
from jax import lax
from jax.experimental import pallas as pl
from jax.experimental.pallas import tpu as pltpu
from jax.experimental.pallas import tpu_sc as plsc

NUM_L0, NUM_L1, NUM_L2 = 4, 16, 16
P = NUM_L0 + NUM_L1 + NUM_L2 + 1
B, T, D = 64, 256, 128
S = T * P
N = B * S

T_L0_BASE = 0
T_L1_BASE = 16
T_L2_BASE = 80
T_ACT_BASE = 144
T_ROWS = 153
T_PAD = 256
BIAS_PAD = 40

NC, NS = 2, 16
NW = NC * NS
SW = S // NW
NB = 2
NG = B // NB
H1, H2 = 152, 144


def _build_t_body(l0_ref, l1_ref, l2_ref, act_ref, lvl_ref, patch_ref,
                  t_ref, bias_ref):
    t_ref[T_L0_BASE:T_L1_BASE, :] = l0_ref[...]
    t_ref[T_L1_BASE:T_L2_BASE, :] = l1_ref[...]
    t_ref[T_L2_BASE:T_ACT_BASE, :] = l2_ref[...]
    t_ref[T_ACT_BASE:T_ROWS, :] = act_ref[...]
    t_ref[T_ROWS:T_PAD, :] = jnp.zeros((T_PAD - T_ROWS, D), jnp.float32)
    for p in range(NUM_L0):
        bias_ref[p : p + 1, :] = lvl_ref[0:1, :] + patch_ref[p : p + 1, :]
    for j in range(NUM_L1):
        bias_ref[NUM_L0 + j : NUM_L0 + j + 1, :] = (
            lvl_ref[1:2, :] + patch_ref[j : j + 1, :]
        )
    for j in range(NUM_L2):
        bias_ref[20 + j : 21 + j, :] = lvl_ref[2:3, :] + patch_ref[j : j + 1, :]
    bias_ref[36:37, :] = lvl_ref[3:4, :]
    bias_ref[37:BIAS_PAD, :] = jnp.zeros((BIAS_PAD - 37, D), jnp.float32)


def _build_idx_body(t0_ref, t1_ref, t2_ref, act_ref, out_ref):
    out_ref[:, :, 0:NUM_L0] = t0_ref[...] + T_L0_BASE
    out_ref[:, :, NUM_L0 : NUM_L0 + NUM_L1] = t1_ref[...] + T_L1_BASE
    out_ref[:, :, NUM_L0 + NUM_L1 : P - 1] = t2_ref[...] + T_L2_BASE
    out_ref[:, :, P - 1 : P] = act_ref[...] + T_ACT_BASE


def _sc_body(t_hbm, idx_hbm, pos_hbm, bias_hbm, out_hbm, t_sh,
             bias_v, pos_v, idx0, idx1, idx2, idx3, wk0, wk1,
             is0, is1, is2, is3, gsa0, gsb0, gsa1, gsb1,
             ssa0, ssb0, ssa1, ssb1):
    cid = lax.axis_index("c")
    sid = lax.axis_index("s")
    wid = sid * NC + cid
    s0 = wid * SW

    idxs = (idx0, idx1, idx2, idx3)
    works = (wk0, wk1)
    isems = (is0, is1, is2, is3)
    gsems = ((gsa0, gsb0), (gsa1, gsb1))
    ssems = ((ssa0, ssb0), (ssa1, ssb1))
    HALVES = (((0, 128), (128, 24)), ((152, 104), (256, 40)))
    HOFF = (0, H1)
    HLEN = (H1, H2)

    tr = T_PAD // NS
    st0 = pltpu.async_copy(
        t_hbm.at[pl.ds(sid * tr, tr)], t_sh.at[pl.ds(sid * tr, tr)], is0)
    st1 = pltpu.async_copy(pos_hbm.at[pl.ds(s0, SW)], pos_v, is1)
    st2 = pltpu.async_copy(bias_hbm, bias_v, gsa0)
    st0.wait()
    st1.wait()
    st2.wait()
    plsc.subcore_barrier()

    @plsc.parallel_loop(0, SW, 1, unroll=4)
    def _(i):
        m = lax.rem(i, P)
        for v in range(D // 16):
            sl = pl.ds(v * 16, 16)
            pos_v[i, sl] = pos_v[i, sl] + bias_v[m, sl]

    def issue_idx(c, j):
        pltpu.async_copy(idx_hbm.at[pl.ds(c * S + s0, SW)], idxs[j], isems[j])

    def wait_idx(j):
        pltpu.make_async_copy(
            idx_hbm.at[pl.ds(0, SW)], idxs[j], isems[j]).wait()

    def issue_gather_half(b, h, j):
        for (o, n) in HALVES[h]:
            pltpu.async_copy(
                t_sh.at[idxs[j].at[pl.ds(o, n)]],
                works[b].at[pl.ds(o, n)], gsems[b][h])

    def wait_gather_half(b, h, j):
        for (o, n) in HALVES[h]:
            pltpu.make_async_copy(
                t_sh.at[idxs[j].at[pl.ds(o, n)]],
                works[b].at[pl.ds(o, n)], gsems[b][h]).wait()

    def issue_store_half(b, h, c):
        off = c * S + s0 + HOFF[h]
        pltpu.async_copy(
            works[b].at[pl.ds(HOFF[h], HLEN[h])],
            out_hbm.at[pl.ds(off, HLEN[h])], ssems[b][h])

    def wait_store_half(b, h):
        pltpu.make_async_copy(
            works[b].at[pl.ds(HOFF[h], HLEN[h])],
            out_hbm.at[pl.ds(0, HLEN[h])], ssems[b][h]).wait()

    def add_pos(b, lo, hi):
        @plsc.parallel_loop(lo, hi, 1, unroll=4)
        def _(i):
            for v in range(D // 16):
                sl = pl.ds(v * 16, 16)
                works[b][i, sl] = works[b][i, sl] + pos_v[i, sl]

    for c in range(3):
        issue_idx(c, c)
    for b in range(2):
        wait_idx(b)
        issue_gather_half(b, 0, b)
        issue_gather_half(b, 1, b)

    def outer(gg, _):
        for par in range(2):
            g = gg * 2 + par
            c0 = g * NB
            j0 = (2 * par) % 4
            j1 = (2 * par + 1) % 4
            j2 = (2 * par + 2) % 4
            j3 = (2 * par + 3) % 4

            wait_gather_half(0, 0, j0)

            @pl.when(g > 0)
            def _():
                wait_store_half(1, 0)
                wait_idx(j1)
                issue_gather_half(1, 0, j1)

            @pl.when(c0 + 3 < B)
            def _():
                issue_idx(c0 + 3, j3)

            add_pos(0, 0, H1)
            issue_store_half(0, 0, c0)

            wait_gather_half(0, 1, j0)

            @pl.when(g > 0)
            def _():
                wait_store_half(1, 1)
                issue_gather_half(1, 1, j1)

            add_pos(0, H1, SW)
            issue_store_half(0, 1, c0)

            wait_gather_half(1, 0, j1)

            @pl.when(c0 + 2 < B)
            def _():
                wait_store_half(0, 0)
                wait_idx(j2)
                issue_gather_half(0, 0, j2)

            @pl.when(c0 + 4 < B)
            def _():
                issue_idx(c0 + 4, j0)

            add_pos(1, 0, H1)
            issue_store_half(1, 0, c0 + 1)

            wait_gather_half(1, 1, j1)

            @pl.when(c0 + 2 < B)
            def _():
                wait_store_half(0, 1)
                issue_gather_half(0, 1, j2)

            add_pos(1, H1, SW)
            issue_store_half(1, 1, c0 + 1)

        return 0

    lax.fori_loop(0, NG // 2, outer, 0)

    wait_store_half(0, 0)
    wait_store_half(0, 1)
    wait_store_half(1, 0)
    wait_store_half(1, 1)


def _make_sc_gather():
    return pl.kernel(
        _sc_body,
        out_type=jax.ShapeDtypeStruct((N, D), jnp.float32),
        mesh=plsc.VectorSubcoreMesh(
            core_axis_name="c", subcore_axis_name="s",
            num_cores=NC, num_subcores=NS,
        ),
        scratch_types=[
            pltpu.VMEM_SHARED((T_PAD, D), jnp.float32),
            pltpu.VMEM((BIAS_PAD, D), jnp.float32),
            pltpu.VMEM((SW, D), jnp.float32),
        ]
        + [pltpu.VMEM((SW,), jnp.int32) for _ in range(4)]
        + [pltpu.VMEM((SW, D), jnp.float32) for _ in range(NB)]
        + [pltpu.SemaphoreType.DMA for _ in range(12)],
    )


def kernel(tokens_l0, tokens_l1, tokens_l2, actions, l0_embed, l1_embed,
           l2_embed, act_embed, level_embed, patch_embed, pos_embed):
    t_table, bias = pl.pallas_call(
        _build_t_body,
        out_shape=[
            jax.ShapeDtypeStruct((T_PAD, D), jnp.float32),
            jax.ShapeDtypeStruct((BIAS_PAD, D), jnp.float32),
        ],
    )(l0_embed, l1_embed, l2_embed, act_embed, level_embed, patch_embed)

    bb = 8
    idx = pl.pallas_call(
        _build_idx_body,
        grid=(B // bb,),
        in_specs=[
            pl.BlockSpec((bb, T, NUM_L0), lambda i: (i, 0, 0)),
            pl.BlockSpec((bb, T, NUM_L1), lambda i: (i, 0, 0)),
            pl.BlockSpec((bb, T, NUM_L2), lambda i: (i, 0, 0)),
            pl.BlockSpec((bb, T, 1), lambda i: (i, 0, 0)),
        ],
        out_specs=pl.BlockSpec((bb, T, P), lambda i: (i, 0, 0)),
        out_shape=jax.ShapeDtypeStruct((B, T, P), jnp.int32),
    )(
        tokens_l0.astype(jnp.int32),
        tokens_l1.astype(jnp.int32),
        tokens_l2.astype(jnp.int32),
        actions.astype(jnp.int32).reshape(B, T, 1),
    )

    out = _make_sc_gather()(t_table, idx.reshape(N), pos_embed[:S], bias)
    return out.reshape(B, S, D)
